# Initial kernel scaffold; baseline (speedup 1.0000x reference)
import jax
import jax.numpy as jnp
from jax.experimental import pallas as pl


def kernel(x, gate_w, w_gate, w_up, w_down):
    raise NotImplementedError("write your pallas kernel here")


# speedup vs baseline: 2.3993x; 2.3993x over previous
import functools

import jax
import jax.numpy as jnp
from jax.experimental import pallas as pl
from jax.experimental.pallas import tpu as pltpu

E = 64
K = 2
D = 768
H = 1536
T = 2048


TB = 1024


def _moe_dense_kernel(x_ref, gate_ref, wg_ref, wu_ref, wd_ref, o_ref, r_ref):
    t = pl.program_id(0)
    e = pl.program_id(1)

    @pl.when(e == 0)
    def _router():
        x = x_ref[...]
        scores = jax.lax.dot_general(
            x, gate_ref[...], (((1,), (1,)), ((), ())),
            preferred_element_type=jnp.float32)
        cols = jax.lax.broadcasted_iota(jnp.int32, (TB, E), 1)
        m1 = jnp.max(scores, axis=1, keepdims=True)
        i1 = jnp.argmax(scores, axis=1).reshape(TB, 1)
        masked = jnp.where(cols == i1, -jnp.inf, scores)
        m2 = jnp.max(masked, axis=1, keepdims=True)
        i2 = jnp.argmax(masked, axis=1).reshape(TB, 1)
        e2 = jnp.exp(m2 - m1)
        w1 = 1.0 / (1.0 + e2)
        w2 = e2 / (1.0 + e2)
        r_ref[:, 0:1] = i1.astype(jnp.float32)
        r_ref[:, 1:2] = i2.astype(jnp.float32)
        r_ref[:, 2:3] = w1
        r_ref[:, 3:4] = w2
        o_ref[...] = jnp.zeros_like(o_ref)

    ef = e.astype(jnp.float32)
    i1 = r_ref[:, 0:1]
    i2 = r_ref[:, 1:2]
    w1 = r_ref[:, 2:3]
    w2 = r_ref[:, 3:4]
    wsel = jnp.where(i1 == ef, w1, 0.0) + jnp.where(i2 == ef, w2, 0.0)

    x = x_ref[...]
    g = jax.lax.dot_general(x, wg_ref[0], (((1,), (1,)), ((), ())),
                            preferred_element_type=jnp.float32)
    u = jax.lax.dot_general(x, wu_ref[0], (((1,), (1,)), ((), ())),
                            preferred_element_type=jnp.float32)
    h = (g * jax.nn.sigmoid(g)) * u
    y = jax.lax.dot_general(h, wd_ref[0], (((1,), (1,)), ((), ())),
                            preferred_element_type=jnp.float32)
    o_ref[...] += wsel * y


@functools.partial(jax.jit, static_argnames=("interpret",))
def kernel(x, gate_w, w_gate, w_up, w_down, interpret=False):
    xf = x.reshape(T, D)
    out = pl.pallas_call(
        _moe_dense_kernel,
        grid=(T // TB, E),
        in_specs=[
            pl.BlockSpec((TB, D), lambda t, e: (t, 0)),
            pl.BlockSpec((E, D), lambda t, e: (0, 0)),
            pl.BlockSpec((1, H, D), lambda t, e: (e, 0, 0)),
            pl.BlockSpec((1, H, D), lambda t, e: (e, 0, 0)),
            pl.BlockSpec((1, D, H), lambda t, e: (e, 0, 0)),
        ],
        out_specs=pl.BlockSpec((TB, D), lambda t, e: (t, 0)),
        out_shape=jax.ShapeDtypeStruct((T, D), jnp.float32),
        scratch_shapes=[pltpu.VMEM((TB, 8), jnp.float32)],
        interpret=interpret,
    )(xf, gate_w, w_gate, w_up, w_down)
    return out.reshape(x.shape)


# speedup vs baseline: 6.3251x; 2.6362x over previous
import functools

import jax
import jax.numpy as jnp
from jax.experimental import pallas as pl
from jax.experimental.pallas import tpu as pltpu

E = 64
K = 2
D = 768
H = 1536
T = 2048
NSLOT = K * T
TM = 256
SLOTS_PAD = NSLOT + 7 * E + TM
H2 = H // 2
RB = 256


def _router_kernel(x_ref, gate_ref, pos1_ref, pos2_ref, w1_ref, w2_ref,
                   offs_ref, c_scr):
    x = x_ref[...]
    scores = jax.lax.dot_general(
        x, gate_ref[...], (((1,), (1,)), ((), ())),
        preferred_element_type=jnp.float32)
    cols = jax.lax.broadcasted_iota(jnp.int32, (T, E), 1)
    m1 = jnp.max(scores, axis=1, keepdims=True)
    i1 = jnp.argmax(scores, axis=1).reshape(T, 1)
    masked = jnp.where(cols == i1, -jnp.inf, scores)
    m2 = jnp.max(masked, axis=1, keepdims=True)
    i2 = jnp.argmax(masked, axis=1).reshape(T, 1)
    e2 = jnp.exp(m2 - m1)
    w1_ref[...] = 1.0 / (1.0 + e2)
    w2_ref[...] = e2 / (1.0 + e2)

    r1 = (cols == i1)
    r2 = (cols == i2)
    rsum_bf = r1.astype(jnp.bfloat16) + r2.astype(jnp.bfloat16)

    counts = jnp.sum(rsum_bf.astype(jnp.float32), axis=0, keepdims=True)
    counts_al = jnp.floor((counts + 7.0) * 0.125) * 8.0
    cp2 = jnp.concatenate([counts_al, jnp.zeros((1, E), jnp.float32)], axis=1)
    er = jax.lax.broadcasted_iota(jnp.int32, (2 * E, 2 * E), 0)
    ec = jax.lax.broadcasted_iota(jnp.int32, (2 * E, 2 * E), 1)
    ltri2 = (er < ec).astype(jnp.float32)
    offs128 = jax.lax.dot_general(cp2, ltri2, (((1,), (0,)), ((), ())),
                                  preferred_element_type=jnp.float32)
    offs = offs128[:, :E]

    for r in range(T // RB):
        ir = jax.lax.broadcasted_iota(jnp.int32, (RB, T), 0) + r * RB
        ic = jax.lax.broadcasted_iota(jnp.int32, (RB, T), 1)
        ltb = (ir > ic).astype(jnp.bfloat16)
        cb = jax.lax.dot_general(ltb, rsum_bf, (((1,), (0,)), ((), ())),
                                 preferred_element_type=jnp.float32)
        c_scr[r * RB:(r + 1) * RB, :] = cb

    posb = c_scr[...] + offs
    pos1_ref[...] = jnp.sum(jnp.where(r1, posb, 0.0), axis=1,
                            keepdims=True).astype(jnp.int32)
    pos2_ref[...] = jnp.sum(jnp.where(r2, posb, 0.0), axis=1,
                            keepdims=True).astype(jnp.int32)

    offs_ref[...] = offs128.astype(jnp.int32)


def _router(xf, gate_w):
    return pl.pallas_call(
        _router_kernel,
        out_shape=(
            jax.ShapeDtypeStruct((T, 1), jnp.int32),
            jax.ShapeDtypeStruct((T, 1), jnp.int32),
            jax.ShapeDtypeStruct((T, 1), jnp.float32),
            jax.ShapeDtypeStruct((T, 1), jnp.float32),
            jax.ShapeDtypeStruct((1, 2 * E), jnp.int32),
        ),
        scratch_shapes=[pltpu.VMEM((T, E), jnp.float32)],
    )(xf, gate_w)


def _ffn_kernel(offs_ref, xs_ref, wg_ref, wu_ref, wd_ref, ys_ref):
    e = pl.program_id(0)
    h = pl.program_id(1)
    off = offs_ref[e]
    n = offs_ref[e + 1] - off
    ntiles = jax.lax.div(n + (TM - 1), TM)

    def tile_body(r, carry):
        base = pl.multiple_of(off + r * TM, 8)
        xt = xs_ref[pl.ds(base, TM), :]
        g = jax.lax.dot_general(xt, wg_ref[0], (((1,), (1,)), ((), ())),
                                preferred_element_type=jnp.float32)
        u = jax.lax.dot_general(xt, wu_ref[0], (((1,), (1,)), ((), ())),
                                preferred_element_type=jnp.float32)
        hh = (g * jax.nn.sigmoid(g)) * u
        y = jax.lax.dot_general(hh, wd_ref[0], (((1,), (1,)), ((), ())),
                                preferred_element_type=jnp.float32)
        prev = jnp.where(h == 0, jnp.zeros_like(y), ys_ref[pl.ds(base, TM), :])
        ys_ref[pl.ds(base, TM), :] = prev + y
        return carry

    jax.lax.fori_loop(0, ntiles, tile_body, 0)


def _ffn(offs, xs, w_gate, w_up, w_down):
    grid_spec = pltpu.PrefetchScalarGridSpec(
        num_scalar_prefetch=1,
        grid=(E, 2),
        in_specs=[
            pl.BlockSpec((SLOTS_PAD, D), lambda e, h, offs: (0, 0)),
            pl.BlockSpec((1, H2, D), lambda e, h, offs: (e, h, 0)),
            pl.BlockSpec((1, H2, D), lambda e, h, offs: (e, h, 0)),
            pl.BlockSpec((1, D, H2), lambda e, h, offs: (e, 0, h)),
        ],
        out_specs=pl.BlockSpec((SLOTS_PAD, D), lambda e, h, offs: (0, 0)),
    )
    return pl.pallas_call(
        _ffn_kernel,
        grid_spec=grid_spec,
        out_shape=jax.ShapeDtypeStruct((SLOTS_PAD, D), jnp.float32),
    )(offs, xs, w_gate, w_up, w_down)


@jax.jit
def kernel(x, gate_w, w_gate, w_up, w_down):
    xf = x.reshape(T, D)
    pos1, pos2, w1, w2, offs = _router(xf, gate_w)
    pos1 = pos1.reshape(T)
    pos2 = pos2.reshape(T)
    offs = offs.reshape(2 * E)

    xs = jnp.zeros((SLOTS_PAD, D), jnp.float32)
    xs = xs.at[pos1].set(xf).at[pos2].set(xf)

    ys = _ffn(offs, xs, w_gate, w_up, w_down)

    out = w1.reshape(T, 1) * ys[pos1] + w2.reshape(T, 1) * ys[pos2]
    return out.reshape(x.shape)


# speedup vs baseline: 6.6454x; 1.0506x over previous
import functools

import jax
import jax.numpy as jnp
from jax import lax
from jax.experimental import pallas as pl
from jax.experimental.pallas import tpu as pltpu
from jax.experimental.pallas import tpu_sc as plsc

E = 64
K = 2
D = 768
H = 1536
T = 2048
NSLOT = K * T
TM = 256
SLOTS_PAD = NSLOT + 7 * E + TM
H2 = H // 2
RB = 256


def _router_kernel(x_ref, gate_ref, pos1_ref, pos2_ref, w1_ref, w2_ref,
                   offs_ref, c_scr):
    x = x_ref[...]
    scores = jax.lax.dot_general(
        x, gate_ref[...], (((1,), (1,)), ((), ())),
        preferred_element_type=jnp.float32)
    cols = jax.lax.broadcasted_iota(jnp.int32, (T, E), 1)
    m1 = jnp.max(scores, axis=1, keepdims=True)
    i1 = jnp.argmax(scores, axis=1).reshape(T, 1)
    masked = jnp.where(cols == i1, -jnp.inf, scores)
    m2 = jnp.max(masked, axis=1, keepdims=True)
    i2 = jnp.argmax(masked, axis=1).reshape(T, 1)
    e2 = jnp.exp(m2 - m1)
    ones16 = jnp.ones((1, 16), jnp.float32)
    w1_ref[...] = (1.0 / (1.0 + e2)) * ones16
    w2_ref[...] = (e2 / (1.0 + e2)) * ones16

    r1 = (cols == i1)
    r2 = (cols == i2)
    rsum_bf = r1.astype(jnp.bfloat16) + r2.astype(jnp.bfloat16)

    counts = jnp.sum(rsum_bf.astype(jnp.float32), axis=0, keepdims=True)
    counts_al = jnp.floor((counts + 7.0) * 0.125) * 8.0
    cp2 = jnp.concatenate([counts_al, jnp.zeros((1, E), jnp.float32)], axis=1)
    er = jax.lax.broadcasted_iota(jnp.int32, (2 * E, 2 * E), 0)
    ec = jax.lax.broadcasted_iota(jnp.int32, (2 * E, 2 * E), 1)
    ltri2 = (er < ec).astype(jnp.float32)
    offs128 = jax.lax.dot_general(cp2, ltri2, (((1,), (0,)), ((), ())),
                                  preferred_element_type=jnp.float32)
    offs = offs128[:, :E]

    for r in range(T // RB):
        ir = jax.lax.broadcasted_iota(jnp.int32, (RB, T), 0) + r * RB
        ic = jax.lax.broadcasted_iota(jnp.int32, (RB, T), 1)
        ltb = (ir > ic).astype(jnp.bfloat16)
        cb = jax.lax.dot_general(ltb, rsum_bf, (((1,), (0,)), ((), ())),
                                 preferred_element_type=jnp.float32)
        c_scr[r * RB:(r + 1) * RB, :] = cb

    posb = c_scr[...] + offs
    pos1_ref[...] = jnp.sum(jnp.where(r1, posb, 0.0), axis=1,
                            keepdims=True).astype(jnp.int32)
    pos2_ref[...] = jnp.sum(jnp.where(r2, posb, 0.0), axis=1,
                            keepdims=True).astype(jnp.int32)

    offs_ref[...] = offs128.astype(jnp.int32)


def _router(xf, gate_w):
    return pl.pallas_call(
        _router_kernel,
        out_shape=(
            jax.ShapeDtypeStruct((T, 1), jnp.int32),
            jax.ShapeDtypeStruct((T, 1), jnp.int32),
            jax.ShapeDtypeStruct((T, 16), jnp.float32),
            jax.ShapeDtypeStruct((T, 16), jnp.float32),
            jax.ShapeDtypeStruct((1, 2 * E), jnp.int32),
        ),
        scratch_shapes=[pltpu.VMEM((T, E), jnp.float32)],
    )(xf, gate_w)


NC = 2
NS = 16
NW = NC * NS
TPW = T // NW
_SC_MESH = dict(core_axis_name="c", subcore_axis_name="s")


def _dispatch_body(x_hbm, pos1_hbm, pos2_hbm,
                   xs_hbm, rows_v, idx1_v, idx2_v, sem):
    wid = lax.axis_index("s") * NC + lax.axis_index("c")
    base = wid * TPW
    pltpu.sync_copy(x_hbm.at[pl.ds(base, TPW)], rows_v)
    pltpu.sync_copy(pos1_hbm.at[pl.ds(base, TPW)], idx1_v)
    pltpu.sync_copy(pos2_hbm.at[pl.ds(base, TPW)], idx2_v)
    pltpu.async_copy(rows_v, xs_hbm.at[idx1_v], sem).wait()
    pltpu.async_copy(rows_v, xs_hbm.at[idx2_v], sem).wait()


def _dispatch(xf, pos1, pos2):
    return pl.kernel(
        _dispatch_body,
        out_type=jax.ShapeDtypeStruct((SLOTS_PAD, D), jnp.float32),
        mesh=plsc.VectorSubcoreMesh(**_SC_MESH),
        scratch_types=[
            pltpu.VMEM((TPW, D), jnp.float32),
            pltpu.VMEM((TPW,), jnp.int32),
            pltpu.VMEM((TPW,), jnp.int32),
            pltpu.SemaphoreType.DMA,
        ],
    )(xf, pos1, pos2)


def _combine_body(ys_hbm, pos1_hbm, pos2_hbm, w1_hbm, w2_hbm, out_hbm,
                  idx1_v, idx2_v, w1_v, w2_v, buf1, buf2, sem):
    wid = lax.axis_index("s") * NC + lax.axis_index("c")
    base = wid * TPW
    pltpu.sync_copy(pos1_hbm.at[pl.ds(base, TPW)], idx1_v)
    pltpu.sync_copy(pos2_hbm.at[pl.ds(base, TPW)], idx2_v)
    pltpu.sync_copy(w1_hbm.at[pl.ds(base, TPW)], w1_v)
    pltpu.sync_copy(w2_hbm.at[pl.ds(base, TPW)], w2_v)
    pltpu.async_copy(ys_hbm.at[idx1_v], buf1, sem).wait()
    pltpu.async_copy(ys_hbm.at[idx2_v], buf2, sem).wait()

    def trow(t, carry):
        wa = w1_v[t, :]
        wb = w2_v[t, :]

        def tcol(c, carry2):
            o = pl.ds(c * 16, 16)
            buf1[t, o] = wa * buf1[t, o] + wb * buf2[t, o]
            return carry2

        lax.fori_loop(0, D // 16, tcol, 0)
        return carry

    lax.fori_loop(0, TPW, trow, 0)
    pltpu.sync_copy(buf1, out_hbm.at[pl.ds(base, TPW)])


def _combine(ys, pos1, pos2, w1, w2):
    return pl.kernel(
        _combine_body,
        out_type=jax.ShapeDtypeStruct((T, D), jnp.float32),
        mesh=plsc.VectorSubcoreMesh(**_SC_MESH),
        scratch_types=[
            pltpu.VMEM((TPW,), jnp.int32),
            pltpu.VMEM((TPW,), jnp.int32),
            pltpu.VMEM((TPW, 16), jnp.float32),
            pltpu.VMEM((TPW, 16), jnp.float32),
            pltpu.VMEM((TPW, D), jnp.float32),
            pltpu.VMEM((TPW, D), jnp.float32),
            pltpu.SemaphoreType.DMA,
        ],
    )(ys, pos1, pos2, w1, w2)


def _ffn_kernel(offs_ref, xs_ref, wg_ref, wu_ref, wd_ref, ys_ref):
    e = pl.program_id(0)
    h = pl.program_id(1)
    off = offs_ref[e]
    n = offs_ref[e + 1] - off
    ntiles = jax.lax.div(n + (TM - 1), TM)

    def tile_body(r, carry):
        base = pl.multiple_of(off + r * TM, 8)
        xt = xs_ref[pl.ds(base, TM), :]
        g = jax.lax.dot_general(xt, wg_ref[0], (((1,), (1,)), ((), ())),
                                preferred_element_type=jnp.float32)
        u = jax.lax.dot_general(xt, wu_ref[0], (((1,), (1,)), ((), ())),
                                preferred_element_type=jnp.float32)
        hh = (g * jax.nn.sigmoid(g)) * u
        y = jax.lax.dot_general(hh, wd_ref[0], (((1,), (1,)), ((), ())),
                                preferred_element_type=jnp.float32)
        prev = jnp.where(h == 0, jnp.zeros_like(y), ys_ref[pl.ds(base, TM), :])
        ys_ref[pl.ds(base, TM), :] = prev + y
        return carry

    jax.lax.fori_loop(0, ntiles, tile_body, 0)


def _ffn(offs, xs, w_gate, w_up, w_down):
    grid_spec = pltpu.PrefetchScalarGridSpec(
        num_scalar_prefetch=1,
        grid=(E, 2),
        in_specs=[
            pl.BlockSpec((SLOTS_PAD, D), lambda e, h, offs: (0, 0)),
            pl.BlockSpec((1, H2, D), lambda e, h, offs: (e, h, 0)),
            pl.BlockSpec((1, H2, D), lambda e, h, offs: (e, h, 0)),
            pl.BlockSpec((1, D, H2), lambda e, h, offs: (e, 0, h)),
        ],
        out_specs=pl.BlockSpec((SLOTS_PAD, D), lambda e, h, offs: (0, 0)),
    )
    return pl.pallas_call(
        _ffn_kernel,
        grid_spec=grid_spec,
        out_shape=jax.ShapeDtypeStruct((SLOTS_PAD, D), jnp.float32),
    )(offs, xs, w_gate, w_up, w_down)


@jax.jit
def kernel(x, gate_w, w_gate, w_up, w_down):
    xf = x.reshape(T, D)
    pos1, pos2, w1, w2, offs = _router(xf, gate_w)
    pos1 = pos1.reshape(T)
    pos2 = pos2.reshape(T)
    offs = offs.reshape(2 * E)

    xs = _dispatch(xf, pos1, pos2)
    ys = _ffn(offs, xs, w_gate, w_up, w_down)
    out = _combine(ys, pos1, pos2, w1, w2)
    return out.reshape(x.shape)


# speedup vs baseline: 7.4959x; 1.1280x over previous
import jax
import jax.numpy as jnp
from jax import lax
from jax.experimental import pallas as pl
from jax.experimental.pallas import tpu as pltpu
from jax.experimental.pallas import tpu_sc as plsc

E = 64
K = 2
D = 768
H = 1536
T = 2048
NSLOT = K * T
TM = 128
SLOTS_PAD = NSLOT + 7 * E + TM
RB = 256


def _router_kernel(x_ref, gate_ref, pos1_ref, pos2_ref, w1_ref, w2_ref,
                   offs_ref, c_scr):
    x = x_ref[...]
    scores = jax.lax.dot_general(
        x, gate_ref[...], (((1,), (1,)), ((), ())),
        preferred_element_type=jnp.float32)
    cols = jax.lax.broadcasted_iota(jnp.int32, (T, E), 1)
    m1 = jnp.max(scores, axis=1, keepdims=True)
    i1 = jnp.argmax(scores, axis=1).reshape(T, 1)
    masked = jnp.where(cols == i1, -jnp.inf, scores)
    m2 = jnp.max(masked, axis=1, keepdims=True)
    i2 = jnp.argmax(masked, axis=1).reshape(T, 1)
    e2 = jnp.exp(m2 - m1)
    ones16 = jnp.ones((1, 16), jnp.float32)
    w1_ref[...] = (1.0 / (1.0 + e2)) * ones16
    w2_ref[...] = (e2 / (1.0 + e2)) * ones16

    r1 = (cols == i1)
    r2 = (cols == i2)
    rsum_bf = r1.astype(jnp.bfloat16) + r2.astype(jnp.bfloat16)

    counts = jnp.sum(rsum_bf.astype(jnp.float32), axis=0, keepdims=True)
    counts_al = jnp.floor((counts + 7.0) * 0.125) * 8.0
    cp2 = jnp.concatenate([counts_al, jnp.zeros((1, E), jnp.float32)], axis=1)
    er = jax.lax.broadcasted_iota(jnp.int32, (2 * E, 2 * E), 0)
    ec = jax.lax.broadcasted_iota(jnp.int32, (2 * E, 2 * E), 1)
    ltri2 = (er < ec).astype(jnp.float32)
    offs128 = jax.lax.dot_general(cp2, ltri2, (((1,), (0,)), ((), ())),
                                  preferred_element_type=jnp.float32)
    offs = offs128[:, :E]

    for r in range(T // RB):
        ir = jax.lax.broadcasted_iota(jnp.int32, (RB, T), 0) + r * RB
        ic = jax.lax.broadcasted_iota(jnp.int32, (RB, T), 1)
        ltb = (ir > ic).astype(jnp.bfloat16)
        cb = jax.lax.dot_general(ltb, rsum_bf, (((1,), (0,)), ((), ())),
                                 preferred_element_type=jnp.float32)
        c_scr[r * RB:(r + 1) * RB, :] = cb

    posb = c_scr[...] + offs
    pos1_ref[...] = jnp.sum(jnp.where(r1, posb, 0.0), axis=1,
                            keepdims=True).astype(jnp.int32)
    pos2_ref[...] = jnp.sum(jnp.where(r2, posb, 0.0), axis=1,
                            keepdims=True).astype(jnp.int32)

    offs_ref[...] = offs128.astype(jnp.int32)


def _router(xf, gate_w):
    return pl.pallas_call(
        _router_kernel,
        out_shape=(
            jax.ShapeDtypeStruct((T, 1), jnp.int32),
            jax.ShapeDtypeStruct((T, 1), jnp.int32),
            jax.ShapeDtypeStruct((T, 16), jnp.float32),
            jax.ShapeDtypeStruct((T, 16), jnp.float32),
            jax.ShapeDtypeStruct((1, 2 * E), jnp.int32),
        ),
        scratch_shapes=[pltpu.VMEM((T, E), jnp.float32)],
    )(xf, gate_w)


NC = 2
NS = 16
NW = NC * NS
TPW = T // NW
_SC_MESH = dict(core_axis_name="c", subcore_axis_name="s")


def _dispatch_body(x_hbm, pos1_hbm, pos2_hbm, xs_hbm, rows_v,
                   idx1a_v, idx1b_v, idx2a_v, idx2b_v, sem):
    wid = lax.axis_index("s") * NC + lax.axis_index("c")
    base = wid * TPW
    HW = TPW // 2
    l0 = pltpu.async_copy(x_hbm.at[pl.ds(base, HW)], rows_v.at[pl.ds(0, HW)],
                          sem)
    l1 = pltpu.async_copy(pos1_hbm.at[pl.ds(base, HW)], idx1a_v, sem)
    l2 = pltpu.async_copy(pos2_hbm.at[pl.ds(base, HW)], idx2a_v, sem)
    l3 = pltpu.async_copy(pos1_hbm.at[pl.ds(base + HW, HW)], idx1b_v, sem)
    l4 = pltpu.async_copy(pos2_hbm.at[pl.ds(base + HW, HW)], idx2b_v, sem)
    l5 = pltpu.async_copy(x_hbm.at[pl.ds(base + HW, HW)],
                          rows_v.at[pl.ds(HW, HW)], sem)
    l0.wait()
    l1.wait()
    l2.wait()
    d1a = pltpu.async_copy(rows_v.at[pl.ds(0, HW)], xs_hbm.at[idx1a_v], sem)
    d2a = pltpu.async_copy(rows_v.at[pl.ds(0, HW)], xs_hbm.at[idx2a_v], sem)
    l3.wait()
    l4.wait()
    l5.wait()
    d1b = pltpu.async_copy(rows_v.at[pl.ds(HW, HW)], xs_hbm.at[idx1b_v], sem)
    d2b = pltpu.async_copy(rows_v.at[pl.ds(HW, HW)], xs_hbm.at[idx2b_v], sem)
    d1a.wait()
    d2a.wait()
    d1b.wait()
    d2b.wait()


def _dispatch(xf, pos1, pos2):
    return pl.kernel(
        _dispatch_body,
        out_type=jax.ShapeDtypeStruct((SLOTS_PAD, D), jnp.float32),
        mesh=plsc.VectorSubcoreMesh(**_SC_MESH),
        scratch_types=[
            pltpu.VMEM((TPW, D), jnp.float32),
            pltpu.VMEM((TPW // 2,), jnp.int32),
            pltpu.VMEM((TPW // 2,), jnp.int32),
            pltpu.VMEM((TPW // 2,), jnp.int32),
            pltpu.VMEM((TPW // 2,), jnp.int32),
            pltpu.SemaphoreType.DMA,
        ],
    )(xf, pos1, pos2)


def _combine_body(ys_hbm, pos1_hbm, pos2_hbm, w1_hbm, w2_hbm, out_hbm,
                  idx1_v, idx2_v, w1_v, w2_v, buf1, buf2, sem):
    wid = lax.axis_index("s") * NC + lax.axis_index("c")
    base = wid * TPW
    l1 = pltpu.async_copy(pos1_hbm.at[pl.ds(base, TPW)], idx1_v, sem)
    l2 = pltpu.async_copy(pos2_hbm.at[pl.ds(base, TPW)], idx2_v, sem)
    l3 = pltpu.async_copy(w1_hbm.at[pl.ds(base, TPW)], w1_v, sem)
    l4 = pltpu.async_copy(w2_hbm.at[pl.ds(base, TPW)], w2_v, sem)
    l1.wait()
    l2.wait()
    l3.wait()
    l4.wait()
    HW = TPW // 2
    g1a = pltpu.async_copy(ys_hbm.at[idx1_v.at[pl.ds(0, HW)]],
                           buf1.at[pl.ds(0, HW)], sem)
    g2a = pltpu.async_copy(ys_hbm.at[idx2_v.at[pl.ds(0, HW)]],
                           buf2.at[pl.ds(0, HW)], sem)
    g1b = pltpu.async_copy(ys_hbm.at[idx1_v.at[pl.ds(HW, HW)]],
                           buf1.at[pl.ds(HW, HW)], sem)
    g2b = pltpu.async_copy(ys_hbm.at[idx2_v.at[pl.ds(HW, HW)]],
                           buf2.at[pl.ds(HW, HW)], sem)
    g1a.wait()
    g2a.wait()

    @plsc.parallel_loop(0, HW, step=1)
    def _trow_a(t):
        wa = w1_v[t, :]
        wb = w2_v[t, :]
        for c in range(D // 16):
            o = pl.ds(c * 16, 16)
            buf1[t, o] = wa * buf1[t, o] + wb * buf2[t, o]

    oa = pltpu.async_copy(buf1.at[pl.ds(0, HW)],
                          out_hbm.at[pl.ds(base, HW)], sem)
    g1b.wait()
    g2b.wait()

    @plsc.parallel_loop(HW, TPW, step=1)
    def _trow_b(t):
        wa = w1_v[t, :]
        wb = w2_v[t, :]
        for c in range(D // 16):
            o = pl.ds(c * 16, 16)
            buf1[t, o] = wa * buf1[t, o] + wb * buf2[t, o]

    oa.wait()
    pltpu.sync_copy(buf1.at[pl.ds(HW, HW)], out_hbm.at[pl.ds(base + HW, HW)])


def _combine(ys, pos1, pos2, w1, w2):
    return pl.kernel(
        _combine_body,
        out_type=jax.ShapeDtypeStruct((T, D), jnp.float32),
        mesh=plsc.VectorSubcoreMesh(**_SC_MESH),
        scratch_types=[
            pltpu.VMEM((TPW,), jnp.int32),
            pltpu.VMEM((TPW,), jnp.int32),
            pltpu.VMEM((TPW, 16), jnp.float32),
            pltpu.VMEM((TPW, 16), jnp.float32),
            pltpu.VMEM((TPW, D), jnp.float32),
            pltpu.VMEM((TPW, D), jnp.float32),
            pltpu.SemaphoreType.DMA,
        ],
    )(ys, pos1, pos2, w1, w2)


def _ffn_kernel(offs_ref, xs_ref, wg_ref, wu_ref, wd_ref, ys_ref):
    e = pl.program_id(0)
    off = offs_ref[e]
    n = offs_ref[e + 1] - off
    ntiles = jax.lax.div(n + (TM - 1), TM)

    def tile_body(r, carry):
        base = pl.multiple_of(off + r * TM, 8)
        xt = xs_ref[pl.ds(base, TM), :]
        g = jax.lax.dot_general(xt, wg_ref[0], (((1,), (1,)), ((), ())),
                                preferred_element_type=jnp.float32)
        u = jax.lax.dot_general(xt, wu_ref[0], (((1,), (1,)), ((), ())),
                                preferred_element_type=jnp.float32)
        hh = (g * jax.nn.sigmoid(g)) * u
        y = jax.lax.dot_general(hh, wd_ref[0], (((1,), (1,)), ((), ())),
                                preferred_element_type=jnp.float32)
        ys_ref[pl.ds(base, TM), :] = y
        return carry

    jax.lax.fori_loop(0, ntiles, tile_body, 0)


def _ffn(offs, xs, w_gate, w_up, w_down):
    grid_spec = pltpu.PrefetchScalarGridSpec(
        num_scalar_prefetch=1,
        grid=(E,),
        in_specs=[
            pl.BlockSpec((SLOTS_PAD, D), lambda e, offs: (0, 0)),
            pl.BlockSpec((1, H, D), lambda e, offs: (e, 0, 0)),
            pl.BlockSpec((1, H, D), lambda e, offs: (e, 0, 0)),
            pl.BlockSpec((1, D, H), lambda e, offs: (e, 0, 0)),
        ],
        out_specs=pl.BlockSpec((SLOTS_PAD, D), lambda e, offs: (0, 0)),
    )
    return pl.pallas_call(
        _ffn_kernel,
        grid_spec=grid_spec,
        out_shape=jax.ShapeDtypeStruct((SLOTS_PAD, D), jnp.float32),
    )(offs, xs, w_gate, w_up, w_down)


@jax.jit
def kernel(x, gate_w, w_gate, w_up, w_down):
    xf = x.reshape(T, D)
    pos1, pos2, w1, w2, offs = _router(xf, gate_w)
    pos1 = pos1.reshape(T)
    pos2 = pos2.reshape(T)
    offs = offs.reshape(2 * E)

    xs = _dispatch(xf, pos1, pos2)
    ys = _ffn(offs, xs, w_gate, w_up, w_down)
    out = _combine(ys, pos1, pos2, w1, w2)
    return out.reshape(x.shape)
